# Initial kernel scaffold; baseline (speedup 1.0000x reference)
import jax
import jax.numpy as jnp
from jax.experimental import pallas as pl


def kernel(h, edge_index, bias):
    raise NotImplementedError("write your pallas kernel here")


# speedup vs baseline: 13.1201x; 13.1201x over previous
import functools

import jax
import jax.numpy as jnp
import numpy as np
from jax import lax
from jax.experimental import pallas as pl
from jax.experimental.pallas import tpu as pltpu
from jax.experimental.pallas import tpu_sc as plsc

N_NODES = 10000
D = 128
E = 320000

NC = 2
NS = 16
NW = NC * NS
L = 16

C = 128
NCHUNK = E // C
G = C // L

_mesh = plsc.VectorSubcoreMesh(core_axis_name="c", subcore_axis_name="s")


@functools.partial(
    pl.kernel,
    mesh=_mesh,
    compiler_params=pltpu.CompilerParams(needs_layout_passes=False),
    out_type=jax.ShapeDtypeStruct((E,), jnp.float32),
    scratch_types=[
        pltpu.VMEM((C,), jnp.int32),
        pltpu.VMEM((C,), jnp.int32),
        pltpu.VMEM((C, D), jnp.float32),
        pltpu.VMEM((C, D), jnp.float32),
        pltpu.VMEM((C,), jnp.float32),
        pltpu.VMEM((C,), jnp.float32),
        pltpu.VMEM((C,), jnp.float32),
        pltpu.SemaphoreType.DMA,
    ],
)
def _score_kernel(h_hbm, src_hbm, dst_hbm, bias_hbm, out_hbm,
                  sidx, didx, srows, drows, bsv, bdv, outv, sem):
    wid = lax.axis_index("s") * NC + lax.axis_index("c")
    n_mine = NCHUNK // NW + jnp.where(wid < NCHUNK % NW, 1, 0)

    lane = lax.iota(jnp.int32, L)
    onehot = [lane == e for e in range(L)]

    def chunk_body(t, _):
        off = pl.multiple_of((wid + t * NW) * C, C)
        pltpu.sync_copy(src_hbm.at[pl.ds(off, C)], sidx)
        pltpu.sync_copy(dst_hbm.at[pl.ds(off, C)], didx)
        cp1 = pltpu.async_copy(h_hbm.at[sidx], srows, sem)
        cp2 = pltpu.async_copy(h_hbm.at[didx], drows, sem)
        cp3 = pltpu.async_copy(bias_hbm.at[sidx], bsv, sem)
        cp4 = pltpu.async_copy(bias_hbm.at[didx], bdv, sem)
        cp1.wait()
        cp2.wait()
        cp3.wait()
        cp4.wait()

        def group_body(g, _):
            gb = pl.multiple_of(g * L, L)
            res = bsv[pl.ds(gb, L)] + bdv[pl.ds(gb, L)]
            for e in range(L):
                r = gb + e
                acc = srows[r, pl.ds(0, L)] * drows[r, pl.ds(0, L)]
                for k in range(1, D // L):
                    acc = acc + srows[r, pl.ds(k * L, L)] * drows[r, pl.ds(k * L, L)]
                res = jnp.where(onehot[e], res + jnp.sum(acc), res)
            outv[pl.ds(gb, L)] = res
            return 0

        lax.fori_loop(0, G, group_body, 0)
        pltpu.sync_copy(outv, out_hbm.at[pl.ds(off, C)])
        return 0

    lax.fori_loop(0, n_mine, chunk_body, 0)


def kernel(h, edge_index, bias):
    src = edge_index[0].astype(jnp.int32)
    dst = edge_index[1].astype(jnp.int32)
    return _score_kernel(h, src, dst, bias)


# speedup vs baseline: 24.9672x; 1.9030x over previous
import functools

import jax
import jax.numpy as jnp
from jax import lax
from jax.experimental import pallas as pl
from jax.experimental.pallas import tpu as pltpu
from jax.experimental.pallas import tpu_sc as plsc

N_NODES = 10000
D = 128
E = 320000

NC = 2
NS = 16
NW = NC * NS
L = 16

EPW = E // NW
C = 80
NCHUNK_W = EPW // C
G = C // L

_mesh = plsc.VectorSubcoreMesh(core_axis_name="c", subcore_axis_name="s")


@functools.partial(
    pl.kernel,
    mesh=_mesh,
    compiler_params=pltpu.CompilerParams(needs_layout_passes=False),
    out_type=jax.ShapeDtypeStruct((E,), jnp.float32),
    scratch_types=[
        pltpu.VMEM((EPW,), jnp.int32),
        pltpu.VMEM((EPW,), jnp.int32),
        pltpu.VMEM((2, C, D), jnp.float32),
        pltpu.VMEM((2, C, D), jnp.float32),
        pltpu.VMEM((2, C), jnp.float32),
        pltpu.VMEM((2, C), jnp.float32),
        pltpu.VMEM((C,), jnp.float32),
        pltpu.SemaphoreType.DMA,
        pltpu.SemaphoreType.DMA,
    ],
)
def _score_kernel(h_hbm, src_hbm, dst_hbm, bias_hbm, out_hbm,
                  sidx, didx, srows, drows, bsv, bdv, outv, sem0, sem1):
    wid = lax.axis_index("s") * NC + lax.axis_index("c")
    base = wid * EPW
    pltpu.sync_copy(src_hbm.at[pl.ds(base, EPW)], sidx)
    pltpu.sync_copy(dst_hbm.at[pl.ds(base, EPW)], didx)
    sems = (sem0, sem1)

    def issue(t, b):
        off = pl.multiple_of(t * C, C)
        sem = sems[b]
        si = sidx.at[pl.ds(off, C)]
        di = didx.at[pl.ds(off, C)]
        pltpu.async_copy(h_hbm.at[si], srows.at[b], sem)
        pltpu.async_copy(h_hbm.at[di], drows.at[b], sem)
        pltpu.async_copy(bias_hbm.at[si], bsv.at[b], sem)
        pltpu.async_copy(bias_hbm.at[di], bdv.at[b], sem)

    def drain(b):
        sem = sems[b]
        pltpu.make_async_copy(h_hbm.at[pl.ds(0, C)], srows.at[b], sem).wait()
        pltpu.make_async_copy(h_hbm.at[pl.ds(0, C)], drows.at[b], sem).wait()
        pltpu.make_async_copy(bias_hbm.at[pl.ds(0, C)], bsv.at[b], sem).wait()
        pltpu.make_async_copy(bias_hbm.at[pl.ds(0, C)], bdv.at[b], sem).wait()

    lane = lax.iota(jnp.int32, L)
    onehot = [lane == e for e in range(L)]

    def compute(t, b):
        drain(b)

        def group_body(g, _):
            gb = pl.multiple_of(g * L, L)
            res = bsv[b, pl.ds(gb, L)] + bdv[b, pl.ds(gb, L)]
            for e in range(L):
                r = gb + e
                acc = srows[b, r, pl.ds(0, L)] * drows[b, r, pl.ds(0, L)]
                for k in range(1, D // L):
                    acc = acc + srows[b, r, pl.ds(k * L, L)] * drows[b, r, pl.ds(k * L, L)]
                res = jnp.where(onehot[e], res + jnp.sum(acc), res)
            outv[pl.ds(gb, L)] = res
            return 0

        lax.fori_loop(0, G, group_body, 0)
        pltpu.sync_copy(outv, out_hbm.at[pl.ds(base + t * C, C)])

    issue(0, 0)

    def pair_body(i, _):
        t = i * 2
        issue(t + 1, 1)
        compute(t, 0)
        issue(t + 2, 0)
        compute(t + 1, 1)
        return 0

    lax.fori_loop(0, (NCHUNK_W - 1) // 2, pair_body, 0)
    compute(NCHUNK_W - 1, 0)


def kernel(h, edge_index, bias):
    src = edge_index[0].astype(jnp.int32)
    dst = edge_index[1].astype(jnp.int32)
    return _score_kernel(h, src, dst, bias)
